# baseline (speedup 1.0000x reference)
import jax
import jax.numpy as jnp
from jax.experimental import pallas as pl
from jax.experimental.pallas import tpu as pltpu

_S = 56
_BATCH = 256
_D = 30
_CORES = 2
_BLOCK_S1 = 1
_STEPS = _S // (_CORES * _BLOCK_S1)
_LAMBDA_COORD = 5.0
_LAMBDA_NOOBJ = 0.5


def _plane_sq_diff(p_ref, t_ref, s1, ch):
    e = p_ref[s1, ch, :, :] - t_ref[s1, ch, :, :]
    return e * e


def _loss_kernel(p_ref, t_ref, o_ref):
    j = pl.program_id(1)

    acc = jnp.zeros((_S, _BATCH), jnp.float32)
    for s1 in range(_BLOCK_S1):
        x = jnp.zeros((_S, _BATCH), jnp.float32)
        for ch in (0, 1):
            x = x + _LAMBDA_COORD * _plane_sq_diff(p_ref, t_ref, s1, ch)
        for ch in (2, 3):
            p = p_ref[s1, ch, :, :]
            t = t_ref[s1, ch, :, :]
            pt = p * t
            s = pt * jax.lax.rsqrt(pt + 1e-20)
            x = x + _LAMBDA_COORD * (p + t - (s + s))
        for ch in range(5, _D):
            x = x + _plane_sq_diff(p_ref, t_ref, s1, ch)
        conf = _plane_sq_diff(p_ref, t_ref, s1, 4)
        x = x + _LAMBDA_NOOBJ * conf
        obj = (t_ref[s1, 4, :, :] == 1.0).astype(jnp.float32)
        acc = acc + obj * x + _LAMBDA_NOOBJ * conf

    partial = jnp.sum(acc, axis=0, keepdims=True) * (1.0 / _BATCH)

    @pl.when(j == 0)
    def _init():
        o_ref[...] = jnp.zeros_like(o_ref)

    o_ref[0] += partial


def kernel(predictions, target):
    pt_ = jnp.transpose(predictions, (1, 3, 2, 0))
    tt_ = jnp.transpose(target, (1, 3, 2, 0))

    in_spec = pl.BlockSpec(
        (_BLOCK_S1, _D, _S, _BATCH), lambda i, j: (i * _STEPS + j, 0, 0, 0))
    out_spec = pl.BlockSpec((1, 1, _BATCH), lambda i, j: (i, 0, 0))

    partials = pl.pallas_call(
        _loss_kernel,
        grid=(_CORES, _STEPS),
        in_specs=[in_spec, in_spec],
        out_specs=out_spec,
        out_shape=jax.ShapeDtypeStruct((_CORES, 1, _BATCH), jnp.float32),
        compiler_params=pltpu.CompilerParams(
            dimension_semantics=("parallel", "arbitrary")),
        name="yolo_loss",
    )(pt_, tt_)

    return jnp.sum(partials)


# speedup vs baseline: 1.1717x; 1.1717x over previous
import jax
import jax.numpy as jnp
from jax.experimental import pallas as pl
from jax.experimental.pallas import tpu as pltpu

_S = 56
_BATCH = 256
_D = 30
_CORES = 2
_BLOCK_S1 = 2
_STEPS = _S // (_CORES * _BLOCK_S1)
_LAMBDA_COORD = 5.0
_LAMBDA_NOOBJ = 0.5


def _plane_sq_diff(p_ref, t_ref, s1, ch):
    e = p_ref[s1, ch, :, :] - t_ref[s1, ch, :, :]
    return e * e


def _loss_kernel(p_ref, t_ref, o_ref):
    j = pl.program_id(1)

    acc = jnp.zeros((_S, _BATCH), jnp.float32)
    for s1 in range(_BLOCK_S1):
        x = jnp.zeros((_S, _BATCH), jnp.float32)
        for ch in (0, 1):
            x = x + _LAMBDA_COORD * _plane_sq_diff(p_ref, t_ref, s1, ch)
        for ch in (2, 3):
            p = p_ref[s1, ch, :, :]
            t = t_ref[s1, ch, :, :]
            pt = p * t
            s = pt * jax.lax.rsqrt(pt + 1e-20)
            x = x + _LAMBDA_COORD * (p + t - (s + s))
        for ch in range(5, _D):
            x = x + _plane_sq_diff(p_ref, t_ref, s1, ch)
        conf = _plane_sq_diff(p_ref, t_ref, s1, 4)
        x = x + _LAMBDA_NOOBJ * conf
        obj = (t_ref[s1, 4, :, :] == 1.0).astype(jnp.float32)
        acc = acc + obj * x + _LAMBDA_NOOBJ * conf

    partial = jnp.sum(acc, axis=0, keepdims=True) * (1.0 / _BATCH)

    @pl.when(j == 0)
    def _init():
        o_ref[...] = jnp.zeros_like(o_ref)

    o_ref[0] += partial


def kernel(predictions, target):
    pt_ = jnp.transpose(predictions, (1, 3, 2, 0))
    tt_ = jnp.transpose(target, (1, 3, 2, 0))

    in_spec = pl.BlockSpec(
        (_BLOCK_S1, _D, _S, _BATCH), lambda i, j: (j * _CORES + i, 0, 0, 0))
    out_spec = pl.BlockSpec((1, 1, _BATCH), lambda i, j: (i, 0, 0))

    partials = pl.pallas_call(
        _loss_kernel,
        grid=(_CORES, _STEPS),
        in_specs=[in_spec, in_spec],
        out_specs=out_spec,
        out_shape=jax.ShapeDtypeStruct((_CORES, 1, _BATCH), jnp.float32),
        compiler_params=pltpu.CompilerParams(
            dimension_semantics=("parallel", "arbitrary")),
        name="yolo_loss",
    )(pt_, tt_)

    return jnp.sum(partials)


# speedup vs baseline: 1.1719x; 1.0002x over previous
import jax
import jax.numpy as jnp
from jax.experimental import pallas as pl
from jax.experimental.pallas import tpu as pltpu

_S = 56
_BATCH = 256
_D = 30
_CORES = 2
_BLOCK_S1 = 2
_STEPS = _S // (_CORES * _BLOCK_S1)
_LAMBDA_COORD = 5.0
_LAMBDA_NOOBJ = 0.5


def _plane_sq_diff(p_ref, t_ref, s1, ch):
    e = p_ref[s1, ch, :, :] - t_ref[s1, ch, :, :]
    return e * e


def _loss_kernel(p_ref, t_ref, o_ref):
    j = pl.program_id(1)

    acc = jnp.zeros((_S, _BATCH), jnp.float32)
    for s1 in range(_BLOCK_S1):
        x = jnp.zeros((_S, _BATCH), jnp.float32)
        for ch in (0, 1):
            x = x + _LAMBDA_COORD * _plane_sq_diff(p_ref, t_ref, s1, ch)
        for ch in (2, 3):
            p = p_ref[s1, ch, :, :]
            t = t_ref[s1, ch, :, :]
            pt = p * t
            s = pt * jax.lax.rsqrt(pt + 1e-20)
            x = x + _LAMBDA_COORD * (p + t - (s + s))
        for ch in range(5, _D):
            x = x + _plane_sq_diff(p_ref, t_ref, s1, ch)
        conf = _plane_sq_diff(p_ref, t_ref, s1, 4)
        x = x + _LAMBDA_NOOBJ * conf
        obj = (t_ref[s1, 4, :, :] == 1.0).astype(jnp.float32)
        acc = acc + obj * x + _LAMBDA_NOOBJ * conf

    partial = jnp.sum(acc, axis=0, keepdims=True) * (1.0 / _BATCH)

    @pl.when(j == 0)
    def _init():
        o_ref[...] = jnp.zeros_like(o_ref)

    o_ref[0] += partial


def kernel(predictions, target):
    pt_ = jnp.transpose(predictions, (1, 3, 2, 0))
    tt_ = jnp.transpose(target, (1, 3, 2, 0))

    in_spec = pl.BlockSpec(
        (_BLOCK_S1, _D, _S, _BATCH), lambda i, j: (i * _STEPS + j, 0, 0, 0))
    out_spec = pl.BlockSpec((1, 1, _BATCH), lambda i, j: (i, 0, 0))

    partials = pl.pallas_call(
        _loss_kernel,
        grid=(_CORES, _STEPS),
        in_specs=[in_spec, in_spec],
        out_specs=out_spec,
        out_shape=jax.ShapeDtypeStruct((_CORES, 1, _BATCH), jnp.float32),
        compiler_params=pltpu.CompilerParams(
            dimension_semantics=("parallel", "arbitrary")),
        name="yolo_loss",
    )(pt_, tt_)

    return jnp.sum(partials)
